# baseline (speedup 1.0000x reference)
import functools

import jax
import jax.numpy as jnp
from jax import lax
from jax.experimental import pallas as pl
from jax.experimental.pallas import tpu as pltpu
from jax.experimental.pallas import tpu_sc as plsc

_NC = 2
_NS = 16
_NW = _NC * _NS
_L = 16


def _sc_hilbert_gather(B, C, P, S):
    N = C * S
    b_per_w = B // _NW
    mesh = plsc.VectorSubcoreMesh(core_axis_name="c", subcore_axis_name="s")

    @functools.partial(
        pl.kernel,
        out_type=jax.ShapeDtypeStruct((B, N), jnp.float32),
        mesh=mesh,
        scratch_types=[
            pltpu.VMEM((S,), jnp.int32),
            pltpu.VMEM((N,), jnp.int32),
            pltpu.VMEM((N,), jnp.float32),
            pltpu.VMEM((N,), jnp.float32),
        ],
        compiler_params=pltpu.CompilerParams(needs_layout_passes=False),
    )
    def k(x_hbm, idx_hbm, out_hbm, idx_v, gidx_v, inb, outb):
        wid = lax.axis_index("s") * _NC + lax.axis_index("c")
        base_b = wid * b_per_w

        pltpu.sync_copy(idx_hbm, idx_v)

        iota = lax.iota(jnp.int32, _L)

        def build(j, carry):
            jv = j * _L + iota
            s_v = jv // C
            c_v = jv - s_v * C
            sidx = plsc.load_gather(idx_v, [s_v])
            gidx_v[pl.ds(j * _L, _L)] = sidx + c_v * P
            return carry
        lax.fori_loop(0, N // _L, build, 0, unroll=4)

        def body(b, carry):
            pltpu.sync_copy(x_hbm.at[base_b + b], inb)

            def gath(j, carry2):
                g_v = gidx_v[pl.ds(j * _L, _L)]
                outb[pl.ds(j * _L, _L)] = plsc.load_gather(inb, [g_v])
                return carry2
            lax.fori_loop(0, N // _L, gath, 0, unroll=8)

            pltpu.sync_copy(outb, out_hbm.at[base_b + b])
            return carry
        lax.fori_loop(0, b_per_w, body, 0)

    return k


def kernel(x, indices):
    B, C, H, W = x.shape
    P = H * W
    S = indices.shape[0]
    xf = x.reshape(B, C * P)
    idx = indices.astype(jnp.int32)
    out = _sc_hilbert_gather(B, C, P, S)(xf, idx)
    return out.reshape(B, S, C)


# speedup vs baseline: 1.1124x; 1.1124x over previous
import functools

import jax
import jax.numpy as jnp
from jax import lax
from jax.experimental import pallas as pl
from jax.experimental.pallas import tpu as pltpu
from jax.experimental.pallas import tpu_sc as plsc

_NC = 2
_NS = 16
_NW = _NC * _NS
_L = 16


def _sc_hilbert_gather(B, C, P, S):
    N = C * S
    b_per_w = B // _NW
    mesh = plsc.VectorSubcoreMesh(core_axis_name="c", subcore_axis_name="s")

    @functools.partial(
        pl.kernel,
        out_type=jax.ShapeDtypeStruct((B, N), jnp.float32),
        mesh=mesh,
        scratch_types=[
            pltpu.VMEM((S,), jnp.int32),
            pltpu.VMEM((N,), jnp.int32),
            pltpu.VMEM((N,), jnp.float32),
            pltpu.VMEM((N,), jnp.float32),
            pltpu.VMEM((N,), jnp.float32),
            pltpu.VMEM((N,), jnp.float32),
            pltpu.SemaphoreType.DMA,
            pltpu.SemaphoreType.DMA,
            pltpu.SemaphoreType.DMA,
            pltpu.SemaphoreType.DMA,
        ],
        compiler_params=pltpu.CompilerParams(needs_layout_passes=False),
    )
    def k(x_hbm, idx_hbm, out_hbm, idx_v, gidx_v, inb0, inb1, outb0, outb1,
          insem0, insem1, outsem0, outsem1):
        inbufs = (inb0, inb1)
        outbufs = (outb0, outb1)
        insems = (insem0, insem1)
        outsems = (outsem0, outsem1)
        wid = lax.axis_index("s") * _NC + lax.axis_index("c")
        base_b = wid * b_per_w

        pltpu.async_copy(x_hbm.at[base_b], inbufs[0], insems[0])

        pltpu.sync_copy(idx_hbm, idx_v)

        iota = lax.iota(jnp.int32, _L)

        def build(j, carry):
            jv = j * _L + iota
            s_v = jv // C
            c_v = jv - s_v * C
            sidx = plsc.load_gather(idx_v, [s_v])
            gidx_v[pl.ds(j * _L, _L)] = sidx + c_v * P
            return carry
        lax.fori_loop(0, N // _L, build, 0, unroll=4)

        n_pairs = b_per_w // 2

        def body(g, carry):
            for sl in range(2):
                b = g * 2 + sl
                nsl = 1 - sl
                @pl.when(b + 1 < b_per_w)
                def _():
                    pltpu.async_copy(
                        x_hbm.at[base_b + b + 1], inbufs[nsl], insems[nsl])
                pltpu.make_async_copy(
                    x_hbm.at[base_b + b], inbufs[sl], insems[sl]).wait()
                @pl.when(b >= 2)
                def _():
                    pltpu.make_async_copy(
                        outbufs[sl], out_hbm.at[base_b + b], outsems[sl]).wait()

                in_sl = inbufs[sl]
                out_sl = outbufs[sl]

                def gath(j, carry2):
                    g_v = gidx_v[pl.ds(j * _L, _L)]
                    out_sl[pl.ds(j * _L, _L)] = plsc.load_gather(in_sl, [g_v])
                    return carry2
                lax.fori_loop(0, N // _L, gath, 0, unroll=8)

                pltpu.async_copy(out_sl, out_hbm.at[base_b + b], outsems[sl])
            return carry
        lax.fori_loop(0, n_pairs, body, 0)

        for sl in range(2):
            pltpu.make_async_copy(
                outbufs[sl], out_hbm.at[base_b], outsems[sl]).wait()

    return k


def kernel(x, indices):
    B, C, H, W = x.shape
    P = H * W
    S = indices.shape[0]
    xf = x.reshape(B, C * P)
    idx = indices.astype(jnp.int32)
    out = _sc_hilbert_gather(B, C, P, S)(xf, idx)
    return out.reshape(B, S, C)


# speedup vs baseline: 1.9856x; 1.7849x over previous
import functools

import jax
import jax.numpy as jnp
from jax import lax
from jax.experimental import pallas as pl
from jax.experimental.pallas import tpu as pltpu
from jax.experimental.pallas import tpu_sc as plsc

_NC = 2
_NS = 16
_NW = _NC * _NS
_L = 16


def _sc_hilbert_gather(B, C, P, S):
    N = C * S
    b_per_w = B // _NW
    mesh = plsc.VectorSubcoreMesh(core_axis_name="c", subcore_axis_name="s")

    @functools.partial(
        pl.kernel,
        out_type=jax.ShapeDtypeStruct((B, N), jnp.float32),
        mesh=mesh,
        scratch_types=[
            pltpu.VMEM((S,), jnp.int32),
            pltpu.VMEM((N,), jnp.int32),
            pltpu.VMEM((N,), jnp.float32),
            pltpu.VMEM((N,), jnp.float32),
            pltpu.VMEM((N,), jnp.float32),
            pltpu.VMEM((N,), jnp.float32),
            pltpu.SemaphoreType.DMA,
            pltpu.SemaphoreType.DMA,
            pltpu.SemaphoreType.DMA,
            pltpu.SemaphoreType.DMA,
        ],
        compiler_params=pltpu.CompilerParams(needs_layout_passes=False),
    )
    def k(x_hbm, idx_hbm, out_hbm, idx_v, gidx_v, inb0, inb1, outb0, outb1,
          insem0, insem1, outsem0, outsem1):
        inbufs = (inb0, inb1)
        outbufs = (outb0, outb1)
        insems = (insem0, insem1)
        outsems = (outsem0, outsem1)
        wid = lax.axis_index("s") * _NC + lax.axis_index("c")
        base_b = wid * b_per_w

        pltpu.async_copy(x_hbm.at[base_b], inbufs[0], insems[0])

        pltpu.sync_copy(idx_hbm, idx_v)

        iota = lax.iota(jnp.int32, _L)

        @plsc.parallel_loop(0, N // _L, unroll=4)
        def _(j):
            jv = j * _L + iota
            s_v = jv // C
            c_v = jv - s_v * C
            sidx = plsc.load_gather(idx_v, [s_v])
            gidx_v[pl.ds(j * _L, _L)] = sidx + c_v * P

        n_pairs = b_per_w // 2

        def body(g, carry):
            for sl in range(2):
                b = g * 2 + sl
                nsl = 1 - sl
                @pl.when(b + 1 < b_per_w)
                def _():
                    pltpu.async_copy(
                        x_hbm.at[base_b + b + 1], inbufs[nsl], insems[nsl])
                pltpu.make_async_copy(
                    x_hbm.at[base_b + b], inbufs[sl], insems[sl]).wait()
                @pl.when(b >= 2)
                def _():
                    pltpu.make_async_copy(
                        outbufs[sl], out_hbm.at[base_b + b], outsems[sl]).wait()

                in_sl = inbufs[sl]
                out_sl = outbufs[sl]

                @plsc.parallel_loop(0, N // _L, unroll=8)
                def _(j):
                    g_v = gidx_v[pl.ds(j * _L, _L)]
                    out_sl[pl.ds(j * _L, _L)] = plsc.load_gather(in_sl, [g_v])

                pltpu.async_copy(out_sl, out_hbm.at[base_b + b], outsems[sl])
            return carry
        lax.fori_loop(0, n_pairs, body, 0)

        for sl in range(2):
            pltpu.make_async_copy(
                outbufs[sl], out_hbm.at[base_b], outsems[sl]).wait()

    return k


def kernel(x, indices):
    B, C, H, W = x.shape
    P = H * W
    S = indices.shape[0]
    xf = x.reshape(B, C * P)
    idx = indices.astype(jnp.int32)
    out = _sc_hilbert_gather(B, C, P, S)(xf, idx)
    return out.reshape(B, S, C)
